# Initial kernel scaffold; baseline (speedup 1.0000x reference)
import jax
import jax.numpy as jnp
from jax.experimental import pallas as pl


def kernel(x, W, b):
    raise NotImplementedError("write your pallas kernel here")


# speedup vs baseline: 5.9535x; 5.9535x over previous
import jax
import jax.numpy as jnp
from jax import lax
from jax.experimental import pallas as pl
from jax.experimental.pallas import tpu as pltpu

N = 1_000_000
D = 64
K = 1024

RB = 1024
G = (N + RB - 1) // RB


def _bitonic_sort(a, m, ascending):
    k = 2
    while k <= m:
        j = k // 2
        while j >= 1:
            n2 = m // (2 * j)
            ar = a.reshape(n2, 2, j, D)
            lo = ar[:, 0]
            hi = ar[:, 1]
            mn = jnp.minimum(lo, hi)
            mx = jnp.maximum(lo, hi)
            b2 = lax.broadcasted_iota(jnp.int32, (n2, j, D), 0)
            first = ((b2 * (2 * j)) & k) == 0
            if ascending:
                new_lo = jnp.where(first, mn, mx)
                new_hi = jnp.where(first, mx, mn)
            else:
                new_lo = jnp.where(first, mx, mn)
                new_hi = jnp.where(first, mn, mx)
            a = jnp.concatenate(
                [new_lo[:, None, :, :], new_hi[:, None, :, :]], axis=1
            ).reshape(m, D)
            j //= 2
        k *= 2
    return a


def _bitonic_clean_desc(a, m):
    j = m // 2
    while j >= 1:
        n2 = m // (2 * j)
        ar = a.reshape(n2, 2, j, D)
        lo = ar[:, 0]
        hi = ar[:, 1]
        mn = jnp.minimum(lo, hi)
        mx = jnp.maximum(lo, hi)
        a = jnp.concatenate(
            [mx[:, None, :, :], mn[:, None, :, :]], axis=1
        ).reshape(m, D)
        j //= 2
    return a


def _body(x_ref, w_ref, b_ref, out_ref, r_ref):
    i = pl.program_id(0)

    @pl.when(i == 0)
    def _():
        r_ref[...] = jnp.full((RB, D), -jnp.inf, jnp.float32)

    gr = i * RB + lax.broadcasted_iota(jnp.int32, (RB, D), 0)
    y = jnp.where(gr < N, x_ref[...], -jnp.inf)
    y_asc = _bitonic_sort(y, RB, ascending=True)
    merged = jnp.maximum(r_ref[...], y_asc)
    r_ref[...] = _bitonic_clean_desc(merged, RB)

    @pl.when(i == G - 1)
    def _():
        pooled = r_ref[...]
        proj = lax.dot_general(
            pooled, w_ref[...], (((1,), (1,)), ((), ())),
            preferred_element_type=jnp.float32,
        )
        out_ref[...] = proj + b_ref[...]


def kernel(x, W, b):
    return pl.pallas_call(
        _body,
        grid=(G,),
        in_specs=[
            pl.BlockSpec((RB, D), lambda i: (i, 0)),
            pl.BlockSpec((D, D), lambda i: (0, 0)),
            pl.BlockSpec((1, D), lambda i: (0, 0)),
        ],
        out_specs=pl.BlockSpec((K, D), lambda i: (0, 0)),
        out_shape=jax.ShapeDtypeStruct((K, D), jnp.float32),
        scratch_shapes=[pltpu.VMEM((RB, D), jnp.float32)],
    )(x, W, b.reshape(1, D))
